# Initial kernel scaffold; baseline (speedup 1.0000x reference)
import jax
import jax.numpy as jnp
from jax.experimental import pallas as pl


def kernel(feature_nchw, fc_w, fc_b, mlp_w, bn_gamma, bn_beta, bn_mean, bn_var, ln_gamma, ln_beta):
    raise NotImplementedError("write your pallas kernel here")


# speedup vs baseline: 2.1095x; 2.1095x over previous
import jax
import jax.numpy as jnp
from jax.experimental import pallas as pl
from jax.experimental.pallas import tpu as pltpu


def _gfa_onepass_kernel(feat_ref, wfc_ref, wmix_ref, shift_ref, ln_g_ref,
                        ln_b_ref, out_ref):
    feat = feat_ref[0]

    logits = jax.lax.dot_general(
        wfc_ref[...], feat, (((0,), (0,)), ((), ())),
        preferred_element_type=jnp.float32)

    m = jnp.max(logits, axis=-1, keepdims=True)
    e = jnp.exp(logits - m)
    att = e / jnp.sum(e, axis=-1, keepdims=True)

    pooled = jax.lax.dot_general(
        feat, att, (((1,), (1,)), ((), ())),
        preferred_element_type=jnp.float32)

    g = jnp.dot(wmix_ref[...], pooled,
                preferred_element_type=jnp.float32) + shift_ref[...]

    mu = jnp.mean(g, axis=0, keepdims=True)
    var = jnp.mean(jnp.square(g - mu), axis=0, keepdims=True)
    gn = (g - mu) * jax.lax.rsqrt(var + 1e-5)
    gn = jnp.maximum(gn * ln_g_ref[...] + ln_b_ref[...], 0.0)

    out_ref[0] = feat + gn


def kernel(feature_nchw, fc_w, fc_b, mlp_w, bn_gamma, bn_beta, bn_mean,
           bn_var, ln_gamma, ln_beta):
    B, C, N, W = feature_nchw.shape
    assert W == 1
    x = feature_nchw[..., 0]

    bn_scale = bn_gamma * jax.lax.rsqrt(bn_var + 1e-5)
    wmix = (mlp_w.astype(jnp.float32)
            * bn_scale[:, None].astype(jnp.float32))
    shift = (bn_beta - bn_mean * bn_scale).astype(jnp.float32)[:, None]
    wfc = fc_w.astype(jnp.float32)[:, None]
    ln_g = ln_gamma.astype(jnp.float32)[:, None]
    ln_b = ln_beta.astype(jnp.float32)[:, None]

    col = pl.BlockSpec((C, 1), lambda b: (0, 0))
    out = pl.pallas_call(
        _gfa_onepass_kernel,
        out_shape=jax.ShapeDtypeStruct((B, C, N), x.dtype),
        grid=(B,),
        in_specs=[
            pl.BlockSpec((1, C, N), lambda b: (b, 0, 0)),
            col,
            pl.BlockSpec((C, C), lambda b: (0, 0)),
            col,
            col,
            col,
        ],
        out_specs=pl.BlockSpec((1, C, N), lambda b: (b, 0, 0)),
        compiler_params=pltpu.CompilerParams(
            dimension_semantics=("parallel",),
            vmem_limit_bytes=48 * 1024 * 1024,
        ),
    )(x, wfc, wmix, shift, ln_g, ln_b)
    return out[..., None]


# speedup vs baseline: 6.7179x; 3.1846x over previous
import jax
import jax.numpy as jnp
from jax.experimental import pallas as pl
from jax.experimental.pallas import tpu as pltpu


def _gfa_kernel(feat_ref, wfc_ref, wmix_ref, shift_ref, ln_g_ref, ln_b_ref,
                out_ref):
    feat = feat_ref[0]
    wfc = wfc_ref[...][:, :, None]

    logits = jnp.sum(feat * wfc, axis=0)

    m = jnp.max(logits, axis=(0, 1), keepdims=True)
    e = jnp.exp(logits - m)
    att = e / jnp.sum(e, axis=(0, 1), keepdims=True)

    pooled = jnp.sum(feat * att[None], axis=(1, 2))
    pooled = pooled.reshape(feat.shape[0], 1)

    g = jnp.dot(wmix_ref[...], pooled,
                preferred_element_type=jnp.float32) + shift_ref[...]

    mu = jnp.mean(g, axis=0, keepdims=True)
    var = jnp.mean(jnp.square(g - mu), axis=0, keepdims=True)
    gn = (g - mu) * jax.lax.rsqrt(var + 1e-5)
    gn = jnp.maximum(gn * ln_g_ref[...] + ln_b_ref[...], 0.0)

    out_ref[0] = feat + gn[:, :, None]


def kernel(feature_nchw, fc_w, fc_b, mlp_w, bn_gamma, bn_beta, bn_mean,
           bn_var, ln_gamma, ln_beta):
    B, C, N, W = feature_nchw.shape
    assert W == 1 and N % 128 == 0
    K = N // 128
    x = feature_nchw.reshape(B, C, K, 128)

    bn_scale = bn_gamma * jax.lax.rsqrt(bn_var + 1e-5)
    wmix = (mlp_w.astype(jnp.float32)
            * bn_scale[:, None].astype(jnp.float32))
    shift = (bn_beta - bn_mean * bn_scale).astype(jnp.float32)[:, None]
    wfc = fc_w.astype(jnp.float32)[:, None]
    ln_g = ln_gamma.astype(jnp.float32)[:, None]
    ln_b = ln_beta.astype(jnp.float32)[:, None]

    col = pl.BlockSpec((C, 1), lambda b: (0, 0))
    out = pl.pallas_call(
        _gfa_kernel,
        out_shape=jax.ShapeDtypeStruct((B, C, K, 128), x.dtype),
        grid=(B,),
        in_specs=[
            pl.BlockSpec((1, C, K, 128), lambda b: (b, 0, 0, 0)),
            col,
            pl.BlockSpec((C, C), lambda b: (0, 0)),
            col,
            col,
            col,
        ],
        out_specs=pl.BlockSpec((1, C, K, 128), lambda b: (b, 0, 0, 0)),
        compiler_params=pltpu.CompilerParams(
            dimension_semantics=("parallel",),
            vmem_limit_bytes=48 * 1024 * 1024,
        ),
    )(x, wfc, wmix, shift, ln_g, ln_b)
    return out.reshape(B, C, N, 1)
